# speedup vs baseline: 7.2060x; 7.2060x over previous
import functools

import jax
import jax.numpy as jnp
from jax import lax
from jax.experimental import pallas as pl
from jax.experimental.pallas import tpu as pltpu
from jax.experimental.pallas import tpu_sc as plsc

VOCAB = 100002
EMB_DIM = 128
BATCH = 1024
SEQ = 200

_INFO = plsc.get_sparse_core_info()
_NC, _NS, _L = _INFO.num_cores, _INFO.num_subcores, _INFO.num_lanes
_NW = _NC * _NS
_B = BATCH * SEQ
_BPW = _B // _NW
_CHUNK = 640
_NCHUNK = _BPW // _CHUNK


def _body(table_hbm, idx_hbm, out_hbm, mask_hbm, idx_v, mask_v, rows_v, sem):
    wid = lax.axis_index("s") * _NC + lax.axis_index("c")
    base = wid * _BPW

    pltpu.sync_copy(idx_hbm.at[pl.ds(base, _BPW)], idx_v)

    def mask_step(i, _):
        v = idx_v[pl.ds(i * _L, _L)]
        mask_v[pl.ds(i * _L, _L)] = jnp.minimum(v, 1)
        return 0

    lax.fori_loop(0, _BPW // _L, mask_step, 0)
    pltpu.sync_copy(mask_v, mask_hbm.at[pl.ds(base, _BPW)])

    for c in range(_NCHUNK):
        off = base + c * _CHUNK
        ids = idx_v.at[pl.ds(c * _CHUNK, _CHUNK)]
        pltpu.async_copy(table_hbm.at[ids], rows_v, sem).wait()
        pltpu.sync_copy(rows_v, out_hbm.at[pl.ds(off, _CHUNK)])


@jax.jit
def _embed(encoded_flat, table):
    mesh = plsc.VectorSubcoreMesh(core_axis_name="c", subcore_axis_name="s")
    out, mask = pl.kernel(
        _body,
        out_type=(
            jax.ShapeDtypeStruct((_B, EMB_DIM), jnp.float32),
            jax.ShapeDtypeStruct((_B,), jnp.int32),
        ),
        mesh=mesh,
        scratch_types=[
            pltpu.VMEM((_BPW,), jnp.int32),
            pltpu.VMEM((_BPW,), jnp.int32),
            pltpu.VMEM((_CHUNK, EMB_DIM), jnp.float32),
            pltpu.SemaphoreType.DMA,
        ],
    )(table, encoded_flat)
    return out, mask


def kernel(encoded, table):
    encoded_flat = encoded.reshape(_B)
    out, mask = _embed(encoded_flat, table)
    return (out.reshape(BATCH, SEQ, EMB_DIM),
            mask.reshape(BATCH, SEQ),
            encoded)


# speedup vs baseline: 7.3678x; 1.0225x over previous
import functools

import jax
import jax.numpy as jnp
from jax import lax
from jax.experimental import pallas as pl
from jax.experimental.pallas import tpu as pltpu
from jax.experimental.pallas import tpu_sc as plsc

VOCAB = 100002
EMB_DIM = 128
BATCH = 1024
SEQ = 200

_INFO = plsc.get_sparse_core_info()
_NC, _NS, _L = _INFO.num_cores, _INFO.num_subcores, _INFO.num_lanes
_NW = _NC * _NS
_B = BATCH * SEQ
_BPW = _B // _NW
_CHUNK = 400
_NCHUNK = _BPW // _CHUNK


def _body(table_hbm, idx_hbm, out_hbm, mask_hbm,
          idx_v, mask_v, rows0, rows1, g0, g1, w0, w1):
    wid = lax.axis_index("s") * _NC + lax.axis_index("c")
    base = wid * _BPW
    bufs, gsems, wsems = (rows0, rows1), (g0, g1), (w0, w1)

    pltpu.sync_copy(idx_hbm.at[pl.ds(base, _BPW)], idx_v)

    def gather(c):
        ids = idx_v.at[pl.ds(c * _CHUNK, _CHUNK)]
        return pltpu.async_copy(table_hbm.at[ids], bufs[c % 2], gsems[c % 2])

    def writeout(c):
        dst = out_hbm.at[pl.ds(base + c * _CHUNK, _CHUNK)]
        return pltpu.async_copy(bufs[c % 2], dst, wsems[c % 2])

    gh = {0: gather(0)}

    def mask_step(i, _):
        v = idx_v[pl.ds(i * _L, _L)]
        mask_v[pl.ds(i * _L, _L)] = jnp.minimum(v, 1)
        return 0

    lax.fori_loop(0, _BPW // _L, mask_step, 0)
    pltpu.sync_copy(mask_v, mask_hbm.at[pl.ds(base, _BPW)])

    wh = {}
    for c in range(_NCHUNK):
        if c >= 1:
            wh[c - 1].wait()
        gh[c].wait()
        if c + 1 < _NCHUNK:
            gh[c + 1] = gather(c + 1)
        wh[c] = writeout(c)
    wh[_NCHUNK - 1].wait()


@jax.jit
def _embed(encoded_flat, table):
    mesh = plsc.VectorSubcoreMesh(core_axis_name="c", subcore_axis_name="s")
    out, mask = pl.kernel(
        _body,
        out_type=(
            jax.ShapeDtypeStruct((_B, EMB_DIM), jnp.float32),
            jax.ShapeDtypeStruct((_B,), jnp.int32),
        ),
        mesh=mesh,
        scratch_types=[
            pltpu.VMEM((_BPW,), jnp.int32),
            pltpu.VMEM((_BPW,), jnp.int32),
            pltpu.VMEM((_CHUNK, EMB_DIM), jnp.float32),
            pltpu.VMEM((_CHUNK, EMB_DIM), jnp.float32),
            pltpu.SemaphoreType.DMA,
            pltpu.SemaphoreType.DMA,
            pltpu.SemaphoreType.DMA,
            pltpu.SemaphoreType.DMA,
        ],
    )(table, encoded_flat)
    return out, mask


def kernel(encoded, table):
    encoded_flat = encoded.reshape(_B)
    out, mask = _embed(encoded_flat, table)
    return (out.reshape(BATCH, SEQ, EMB_DIM),
            mask.reshape(BATCH, SEQ),
            encoded)


# speedup vs baseline: 7.5769x; 1.0284x over previous
import functools

import jax
import jax.numpy as jnp
from jax import lax
from jax.experimental import pallas as pl
from jax.experimental.pallas import tpu as pltpu
from jax.experimental.pallas import tpu_sc as plsc

VOCAB = 100002
EMB_DIM = 128
BATCH = 1024
SEQ = 200

_INFO = plsc.get_sparse_core_info()
_NC, _NS, _L = _INFO.num_cores, _INFO.num_subcores, _INFO.num_lanes
_NW = _NC * _NS
_B = BATCH * SEQ
_BPW = _B // _NW
_CHUNK = 200
_NBUF = 4
_NCHUNK = _BPW // _CHUNK


def _body(table_hbm, idx_hbm, out_hbm, mask_hbm,
          idx_v, mask_v, bufs, gsems, wsems):
    wid = lax.axis_index("s") * _NC + lax.axis_index("c")
    base = wid * _BPW

    pltpu.sync_copy(idx_hbm.at[pl.ds(base, _BPW)], idx_v)

    def gather(c):
        ids = idx_v.at[pl.ds(c * _CHUNK, _CHUNK)]
        b = c % _NBUF
        return pltpu.async_copy(table_hbm.at[ids], bufs[b], gsems[b])

    def writeout(c):
        dst = out_hbm.at[pl.ds(base + c * _CHUNK, _CHUNK)]
        b = c % _NBUF
        return pltpu.async_copy(bufs[b], dst, wsems[b])

    gh = {c: gather(c) for c in range(_NBUF)}

    def mask_step(i, _):
        v = idx_v[pl.ds(i * _L, _L)]
        mask_v[pl.ds(i * _L, _L)] = jnp.minimum(v, 1)
        return 0

    lax.fori_loop(0, _BPW // _L, mask_step, 0)
    pltpu.sync_copy(mask_v, mask_hbm.at[pl.ds(base, _BPW)])

    wh = {}
    for c in range(_NCHUNK):
        gh[c].wait()
        wh[c] = writeout(c)
        if c + _NBUF < _NCHUNK:
            wh[c].wait()
            gh[c + _NBUF] = gather(c + _NBUF)
    for c in range(max(0, _NCHUNK - _NBUF), _NCHUNK):
        wh[c].wait()


@jax.jit
def _embed(encoded_flat, table):
    mesh = plsc.VectorSubcoreMesh(core_axis_name="c", subcore_axis_name="s")
    out, mask = pl.kernel(
        _body,
        out_type=(
            jax.ShapeDtypeStruct((_B, EMB_DIM), jnp.float32),
            jax.ShapeDtypeStruct((_B,), jnp.int32),
        ),
        mesh=mesh,
        scratch_types=[
            pltpu.VMEM((_BPW,), jnp.int32),
            pltpu.VMEM((_BPW,), jnp.int32),
            [pltpu.VMEM((_CHUNK, EMB_DIM), jnp.float32) for _ in range(_NBUF)],
            [pltpu.SemaphoreType.DMA for _ in range(_NBUF)],
            [pltpu.SemaphoreType.DMA for _ in range(_NBUF)],
        ],
    )(table, encoded_flat)
    return out, mask


def kernel(encoded, table):
    encoded_flat = encoded.reshape(_B)
    out, mask = _embed(encoded_flat, table)
    return (out.reshape(BATCH, SEQ, EMB_DIM),
            mask.reshape(BATCH, SEQ),
            encoded)
